# Initial kernel scaffold; baseline (speedup 1.0000x reference)
import jax
import jax.numpy as jnp
from jax.experimental import pallas as pl


def kernel(x, edge_index, etypes, lin_W0, lin_b0, gru_Wih0, gru_Whh0, gru_bih0, gru_bhh0, bn_g0, bn_b0, lin_W1, lin_b1, gru_Wih1, gru_Whh1, gru_bih1, gru_bhh1, bn_g1, bn_b1):
    raise NotImplementedError("write your pallas kernel here")


# speedup vs baseline: 25.6843x; 25.6843x over previous
import functools

import jax
import jax.numpy as jnp
from jax import lax
from jax.experimental import pallas as pl
from jax.experimental.pallas import tpu as pltpu
from jax.experimental.pallas import tpu_sc as plsc

_N = 10000
_E = 320000
_D = 128
_K = 4
_STEPS = 5

_NC = 2
_NS = 16
_NW = _NC * _NS
_EPW = _E // _NW
_C = 80
_CH = _EPW // _C
_NPAD = 10240
_ZR = _NPAD // _NS

_BN = 1000


def _transform_body(h_ref, w_ref, b_ref, out_ref):
  h = h_ref[...]
  for k in range(_K):
    out_ref[k] = jnp.dot(h, w_ref[k], preferred_element_type=jnp.float32) + b_ref[k][None, :]


def _transform(h, lin_W, lin_b):
  return pl.pallas_call(
      _transform_body,
      grid=(_N // _BN,),
      in_specs=[
          pl.BlockSpec((_BN, _D), lambda i: (i, 0)),
          pl.BlockSpec((_K, _D, _D), lambda i: (0, 0, 0)),
          pl.BlockSpec((_K, _D), lambda i: (0, 0)),
      ],
      out_specs=pl.BlockSpec((_K, _BN, _D), lambda i: (0, i, 0)),
      out_shape=jax.ShapeDtypeStruct((_K, _N, _D), jnp.float32),
  )(h, lin_W, lin_b)


def _gru_math(a, h, wih_t, whh_t, bih, bhh):
  gi = jnp.dot(a, wih_t, preferred_element_type=jnp.float32) + bih
  gh = jnp.dot(h, whh_t, preferred_element_type=jnp.float32) + bhh
  r = jax.nn.sigmoid(gi[:, :_D] + gh[:, :_D])
  z = jax.nn.sigmoid(gi[:, _D:2 * _D] + gh[:, _D:2 * _D])
  n = jnp.tanh(gi[:, 2 * _D:] + r * gh[:, 2 * _D:])
  return (1.0 - z) * n + z * h


def _gru_body(a0_ref, a1_ref, h_ref, wih_ref, whh_ref, bih_ref, bhh_ref, out_ref):
  a = a0_ref[...] + a1_ref[...]
  out_ref[...] = _gru_math(a, h_ref[...], wih_ref[...], whh_ref[...],
                           bih_ref[...], bhh_ref[...])


def _gru_tr_body(a0_ref, a1_ref, h_ref, wih_ref, whh_ref, bih_ref, bhh_ref,
                 w_ref, b_ref, out_ref, wh_ref):
  a = a0_ref[...] + a1_ref[...]
  hn = _gru_math(a, h_ref[...], wih_ref[...], whh_ref[...],
                 bih_ref[...], bhh_ref[...])
  out_ref[...] = hn
  for k in range(_K):
    wh_ref[k] = jnp.dot(hn, w_ref[k], preferred_element_type=jnp.float32) + b_ref[k][None, :]


_row_spec = pl.BlockSpec((_BN, _D), lambda i: (i, 0))
_w3_spec = pl.BlockSpec((_D, 3 * _D), lambda i: (0, 0))
_b3_spec = pl.BlockSpec((1, 3 * _D), lambda i: (0, 0))


def _gru(a0, a1, h, wih_t, whh_t, bih, bhh):
  return pl.pallas_call(
      _gru_body,
      grid=(_N // _BN,),
      in_specs=[_row_spec, _row_spec, _row_spec, _w3_spec, _w3_spec, _b3_spec, _b3_spec],
      out_specs=_row_spec,
      out_shape=jax.ShapeDtypeStruct((_N, _D), jnp.float32),
  )(a0, a1, h, wih_t, whh_t, bih, bhh)


def _gru_transform(a0, a1, h, wih_t, whh_t, bih, bhh, lin_W, lin_b):
  return pl.pallas_call(
      _gru_tr_body,
      grid=(_N // _BN,),
      in_specs=[_row_spec, _row_spec, _row_spec, _w3_spec, _w3_spec, _b3_spec, _b3_spec,
                pl.BlockSpec((_K, _D, _D), lambda i: (0, 0, 0)),
                pl.BlockSpec((_K, _D), lambda i: (0, 0))],
      out_specs=[_row_spec, pl.BlockSpec((_K, _BN, _D), lambda i: (0, i, 0))],
      out_shape=[jax.ShapeDtypeStruct((_N, _D), jnp.float32),
                 jax.ShapeDtypeStruct((_K, _N, _D), jnp.float32)],
  )(a0, a1, h, wih_t, whh_t, bih, bhh, lin_W, lin_b)


def _post_body(h0_ref, h1_ref, g0_ref, b0_ref, g1_ref, b1_ref, out_ref):
  eps = 1e-5
  for idx, (h_ref, g_ref, b_ref) in enumerate(
      ((h0_ref, g0_ref, b0_ref), (h1_ref, g1_ref, b1_ref))):
    hv = jnp.maximum(h_ref[...], 0.0)
    mean = jnp.mean(hv, axis=0, keepdims=True)
    var = jnp.mean(jnp.square(hv - mean), axis=0, keepdims=True)
    y = g_ref[...] * (hv - mean) / jnp.sqrt(var + eps) + b_ref[...]
    out_ref[:, idx * _D:(idx + 1) * _D] = y


def _post(h0, h1, g0, b0, g1, b1):
  full = pl.BlockSpec((_N, _D), lambda: (0, 0))
  one = pl.BlockSpec((1, _D), lambda: (0, 0))
  return pl.pallas_call(
      _post_body,
      in_specs=[full, full, one, one, one, one],
      out_specs=pl.BlockSpec((_N, 2 * _D), lambda: (0, 0)),
      out_shape=jax.ShapeDtypeStruct((_N, 2 * _D), jnp.float32),
  )(h0, h1, g0.reshape(1, _D), b0.reshape(1, _D), g1.reshape(1, _D), b1.reshape(1, _D))


def _sc_scatter_body(wh_hbm, gidx_hbm, dst_hbm, zeros_hbm, out_hbm,
                     idx_v, dst_v, rows_v, acc_sh, sem):
  cid = lax.axis_index("c")
  sid = lax.axis_index("s")
  w = cid * _NS + sid

  pltpu.sync_copy(zeros_hbm.at[pl.ds(sid * _ZR, _ZR)], acc_sh.at[pl.ds(sid * _ZR, _ZR)])
  pltpu.sync_copy(gidx_hbm.at[w], idx_v)
  pltpu.sync_copy(dst_hbm.at[w], dst_v)
  plsc.subcore_barrier()

  def body(j, carry):
    pltpu.async_copy(wh_hbm.at[idx_v.at[j]], rows_v, sem).wait()
    pltpu.sync_copy(rows_v, acc_sh.at[dst_v.at[j]], add=True)
    return carry

  lax.fori_loop(0, _CH, body, 0)
  plsc.subcore_barrier()
  pltpu.sync_copy(acc_sh.at[pl.ds(sid * _ZR, _ZR)],
                  out_hbm.at[pl.ds(cid * _NPAD + sid * _ZR, _ZR)])


@functools.partial(jax.jit, static_argnames=())
def _sc_scatter(wh_flat, gidx3, dst3, zeros):
  mesh = plsc.VectorSubcoreMesh(core_axis_name="c", subcore_axis_name="s")
  return pl.kernel(
      _sc_scatter_body,
      out_type=jax.ShapeDtypeStruct((_NC * _NPAD, _D), jnp.float32),
      mesh=mesh,
      scratch_types=[
          pltpu.VMEM((_CH, _C), jnp.int32),
          pltpu.VMEM((_CH, _C), jnp.int32),
          pltpu.VMEM((_C, _D), jnp.float32),
          pltpu.VMEM_SHARED((_NPAD, _D), jnp.float32),
          pltpu.SemaphoreType.DMA,
      ],
  )(wh_flat, gidx3, dst3, zeros)


def _layer(x, gidx3, dst3, zeros, lin_W, lin_b, wih_t, whh_t, bih, bhh):
  h = x
  wh = _transform(h, lin_W, lin_b)
  for step in range(_STEPS):
    parts = _sc_scatter(wh.reshape(_K * _N, _D), gidx3, dst3, zeros)
    a0 = parts[:_N]
    a1 = parts[_NPAD:_NPAD + _N]
    if step < _STEPS - 1:
      h, wh = _gru_transform(a0, a1, h, wih_t, whh_t, bih, bhh, lin_W, lin_b)
    else:
      h = _gru(a0, a1, h, wih_t, whh_t, bih, bhh)
  return h


def kernel(x, edge_index, etypes,
           lin_W0, lin_b0, gru_Wih0, gru_Whh0, gru_bih0, gru_bhh0, bn_g0, bn_b0,
           lin_W1, lin_b1, gru_Wih1, gru_Whh1, gru_bih1, gru_bhh1, bn_g1, bn_b1):
  src = edge_index[0]
  dst = edge_index[1]
  gidx3 = (etypes.astype(jnp.int32) * _N + src.astype(jnp.int32)).reshape(_NW, _CH, _C)
  dst3 = dst.astype(jnp.int32).reshape(_NW, _CH, _C)
  zeros = jnp.zeros((_NPAD, _D), jnp.float32)

  h0 = _layer(x, gidx3, dst3, zeros, lin_W0, lin_b0,
              gru_Wih0.T, gru_Whh0.T, gru_bih0.reshape(1, -1), gru_bhh0.reshape(1, -1))
  h1 = _layer(x, gidx3, dst3, zeros, lin_W1, lin_b1,
              gru_Wih1.T, gru_Whh1.T, gru_bih1.reshape(1, -1), gru_bhh1.reshape(1, -1))
  return _post(h0, h1, bn_g0, bn_b0, bn_g1, bn_b1)


# speedup vs baseline: 45.9377x; 1.7885x over previous
import functools

import jax
import jax.numpy as jnp
from jax import lax
from jax.experimental import pallas as pl
from jax.experimental.pallas import tpu as pltpu
from jax.experimental.pallas import tpu_sc as plsc

_N = 10000
_E = 320000
_D = 128
_K = 4
_STEPS = 5

_NC = 2
_NS = 16
_NW = _NC * _NS
_C = 128
_CH = 80
_EPW = _C * _CH
_EPAD = _NW * _EPW - _E
_NPAD = 10240
_ZR = _NPAD // _NS

_BN = 1000


def _transform_body(h_ref, w_ref, b_ref, out_ref):
  h = h_ref[...]
  for k in range(_K):
    out_ref[k] = jnp.dot(h, w_ref[k], preferred_element_type=jnp.float32) + b_ref[k][None, :]


def _transform(h, lin_W, lin_b):
  return pl.pallas_call(
      _transform_body,
      grid=(_N // _BN,),
      in_specs=[
          pl.BlockSpec((_BN, _D), lambda i: (i, 0)),
          pl.BlockSpec((_K, _D, _D), lambda i: (0, 0, 0)),
          pl.BlockSpec((_K, _D), lambda i: (0, 0)),
      ],
      out_specs=pl.BlockSpec((_K, _BN, _D), lambda i: (0, i, 0)),
      out_shape=jax.ShapeDtypeStruct((_K, _N, _D), jnp.float32),
  )(h, lin_W, lin_b)


def _gru_math(a, h, wih_t, whh_t, bih, bhh):
  gi = jnp.dot(a, wih_t, preferred_element_type=jnp.float32) + bih
  gh = jnp.dot(h, whh_t, preferred_element_type=jnp.float32) + bhh
  r = jax.nn.sigmoid(gi[:, :_D] + gh[:, :_D])
  z = jax.nn.sigmoid(gi[:, _D:2 * _D] + gh[:, _D:2 * _D])
  n = jnp.tanh(gi[:, 2 * _D:] + r * gh[:, 2 * _D:])
  return (1.0 - z) * n + z * h


def _gru_body(a0_ref, a1_ref, h_ref, wih_ref, whh_ref, bih_ref, bhh_ref, out_ref):
  a = a0_ref[...] + a1_ref[...]
  out_ref[...] = _gru_math(a, h_ref[...], wih_ref[...], whh_ref[...],
                           bih_ref[...], bhh_ref[...])


def _gru_tr_body(a0_ref, a1_ref, h_ref, wih_ref, whh_ref, bih_ref, bhh_ref,
                 w_ref, b_ref, out_ref, wh_ref):
  a = a0_ref[...] + a1_ref[...]
  hn = _gru_math(a, h_ref[...], wih_ref[...], whh_ref[...],
                 bih_ref[...], bhh_ref[...])
  out_ref[...] = hn
  for k in range(_K):
    wh_ref[k] = jnp.dot(hn, w_ref[k], preferred_element_type=jnp.float32) + b_ref[k][None, :]


_row_spec = pl.BlockSpec((_BN, _D), lambda i: (i, 0))
_w3_spec = pl.BlockSpec((_D, 3 * _D), lambda i: (0, 0))
_b3_spec = pl.BlockSpec((1, 3 * _D), lambda i: (0, 0))


def _gru(a0, a1, h, wih_t, whh_t, bih, bhh):
  return pl.pallas_call(
      _gru_body,
      grid=(_N // _BN,),
      in_specs=[_row_spec, _row_spec, _row_spec, _w3_spec, _w3_spec, _b3_spec, _b3_spec],
      out_specs=_row_spec,
      out_shape=jax.ShapeDtypeStruct((_N, _D), jnp.float32),
  )(a0, a1, h, wih_t, whh_t, bih, bhh)


def _gru_transform(a0, a1, h, wih_t, whh_t, bih, bhh, lin_W, lin_b):
  return pl.pallas_call(
      _gru_tr_body,
      grid=(_N // _BN,),
      in_specs=[_row_spec, _row_spec, _row_spec, _w3_spec, _w3_spec, _b3_spec, _b3_spec,
                pl.BlockSpec((_K, _D, _D), lambda i: (0, 0, 0)),
                pl.BlockSpec((_K, _D), lambda i: (0, 0))],
      out_specs=[_row_spec, pl.BlockSpec((_K, _BN, _D), lambda i: (0, i, 0))],
      out_shape=[jax.ShapeDtypeStruct((_N, _D), jnp.float32),
                 jax.ShapeDtypeStruct((_K, _N, _D), jnp.float32)],
  )(a0, a1, h, wih_t, whh_t, bih, bhh, lin_W, lin_b)


def _post_body(h0_ref, h1_ref, g0_ref, b0_ref, g1_ref, b1_ref, out_ref):
  eps = 1e-5
  for idx, (h_ref, g_ref, b_ref) in enumerate(
      ((h0_ref, g0_ref, b0_ref), (h1_ref, g1_ref, b1_ref))):
    hv = jnp.maximum(h_ref[...], 0.0)
    mean = jnp.mean(hv, axis=0, keepdims=True)
    var = jnp.mean(jnp.square(hv - mean), axis=0, keepdims=True)
    y = g_ref[...] * (hv - mean) / jnp.sqrt(var + eps) + b_ref[...]
    out_ref[:, idx * _D:(idx + 1) * _D] = y


def _post(h0, h1, g0, b0, g1, b1):
  full = pl.BlockSpec((_N, _D), lambda: (0, 0))
  one = pl.BlockSpec((1, _D), lambda: (0, 0))
  return pl.pallas_call(
      _post_body,
      in_specs=[full, full, one, one, one, one],
      out_specs=pl.BlockSpec((_N, 2 * _D), lambda: (0, 0)),
      out_shape=jax.ShapeDtypeStruct((_N, 2 * _D), jnp.float32),
  )(h0, h1, g0.reshape(1, _D), b0.reshape(1, _D), g1.reshape(1, _D), b1.reshape(1, _D))


def _unpack(packed_v, j, idx_buf, dst_buf):
  for i in range(_C // 16):
    v = packed_v[pl.ds(j * _C + i * 16, 16)]
    dst_buf[pl.ds(i * 16, 16)] = lax.shift_right_logical(v, 16)
    idx_buf[pl.ds(i * 16, 16)] = lax.bitwise_and(v, 0xFFFF)


def _sc_scatter_body(wh_hbm, packed_hbm, zeros_hbm, out_hbm,
                     packed_v, rows_a, rows_b, ia, da, ib, db,
                     acc_sh, sem_z, sem_a, sem_b):
  cid = lax.axis_index("c")
  sid = lax.axis_index("s")
  w = cid * _NS + sid

  zd = pltpu.async_copy(zeros_hbm.at[pl.ds(sid * _ZR, _ZR)],
                        acc_sh.at[pl.ds(sid * _ZR, _ZR)], sem_z)
  pltpu.sync_copy(packed_hbm.at[w], packed_v)
  zd.wait()
  _unpack(packed_v, 0, ia, da)
  plsc.subcore_barrier()

  pltpu.async_copy(wh_hbm.at[ia], rows_a, sem_a)
  _unpack(packed_v, 1, ib, db)

  def body(t, carry):
    c0 = 2 * t
    pltpu.async_copy(wh_hbm.at[ib], rows_b, sem_b)
    pltpu.make_async_copy(wh_hbm.at[ia], rows_a, sem_a).wait()
    pltpu.sync_copy(rows_a, acc_sh.at[da], add=True)
    _unpack(packed_v, c0 + 2, ia, da)
    pltpu.async_copy(wh_hbm.at[ia], rows_a, sem_a)
    pltpu.make_async_copy(wh_hbm.at[ib], rows_b, sem_b).wait()
    pltpu.sync_copy(rows_b, acc_sh.at[db], add=True)
    _unpack(packed_v, c0 + 3, ib, db)
    return carry

  lax.fori_loop(0, _CH // 2 - 1, body, 0)
  pltpu.async_copy(wh_hbm.at[ib], rows_b, sem_b)
  pltpu.make_async_copy(wh_hbm.at[ia], rows_a, sem_a).wait()
  pltpu.sync_copy(rows_a, acc_sh.at[da], add=True)
  pltpu.make_async_copy(wh_hbm.at[ib], rows_b, sem_b).wait()
  pltpu.sync_copy(rows_b, acc_sh.at[db], add=True)
  plsc.subcore_barrier()
  pltpu.sync_copy(acc_sh.at[pl.ds(sid * _ZR, _ZR)],
                  out_hbm.at[pl.ds(cid * _NPAD + sid * _ZR, _ZR)])


@functools.partial(jax.jit, static_argnames=())
def _sc_scatter(wh_flat, packed, zeros):
  mesh = plsc.VectorSubcoreMesh(core_axis_name="c", subcore_axis_name="s")
  return pl.kernel(
      _sc_scatter_body,
      out_type=jax.ShapeDtypeStruct((_NC * _NPAD, _D), jnp.float32),
      mesh=mesh,
      scratch_types=[
          pltpu.VMEM((_EPW,), jnp.int32),
          pltpu.VMEM((_C, _D), jnp.float32),
          pltpu.VMEM((_C, _D), jnp.float32),
          pltpu.VMEM((_C,), jnp.int32),
          pltpu.VMEM((_C,), jnp.int32),
          pltpu.VMEM((_C,), jnp.int32),
          pltpu.VMEM((_C,), jnp.int32),
          pltpu.VMEM_SHARED((_NPAD, _D), jnp.float32),
          pltpu.SemaphoreType.DMA,
          pltpu.SemaphoreType.DMA,
          pltpu.SemaphoreType.DMA,
      ],
  )(wh_flat, packed, zeros)


def _layer(x, packed, zeros, lin_W, lin_b, wih_t, whh_t, bih, bhh):
  h = x
  wh = _transform(h, lin_W, lin_b)
  for step in range(_STEPS):
    parts = _sc_scatter(wh.reshape(_K * _N, _D), packed, zeros)
    a0 = parts[:_N]
    a1 = parts[_NPAD:_NPAD + _N]
    if step < _STEPS - 1:
      h, wh = _gru_transform(a0, a1, h, wih_t, whh_t, bih, bhh, lin_W, lin_b)
    else:
      h = _gru(a0, a1, h, wih_t, whh_t, bih, bhh)
  return h


def kernel(x, edge_index, etypes,
           lin_W0, lin_b0, gru_Wih0, gru_Whh0, gru_bih0, gru_bhh0, bn_g0, bn_b0,
           lin_W1, lin_b1, gru_Wih1, gru_Whh1, gru_bih1, gru_bhh1, bn_g1, bn_b1):
  src = edge_index[0].astype(jnp.int32)
  dst = edge_index[1].astype(jnp.int32)
  gidx = etypes.astype(jnp.int32) * _N + src
  ar = jnp.arange(_EPAD, dtype=jnp.int32)
  pad_gidx = (ar * 97) % (_K * _N)
  pad_dst = _N + ar % (_NPAD - _N)
  packed = (jnp.concatenate([dst, pad_dst]) << 16) | jnp.concatenate([gidx, pad_gidx])
  packed = packed.reshape(_NW, _EPW)
  zeros = jnp.zeros((_NPAD, _D), jnp.float32)

  h0 = _layer(x, packed, zeros, lin_W0, lin_b0,
              gru_Wih0.T, gru_Whh0.T, gru_bih0.reshape(1, -1), gru_bhh0.reshape(1, -1))
  h1 = _layer(x, packed, zeros, lin_W1, lin_b1,
              gru_Wih1.T, gru_Whh1.T, gru_bih1.reshape(1, -1), gru_bhh1.reshape(1, -1))
  return _post(h0, h1, bn_g0, bn_b0, bn_g1, bn_b1)


# speedup vs baseline: 46.1203x; 1.0040x over previous
import functools

import jax
import jax.numpy as jnp
from jax import lax
from jax.experimental import pallas as pl
from jax.experimental.pallas import tpu as pltpu
from jax.experimental.pallas import tpu_sc as plsc

_N = 10000
_E = 320000
_D = 128
_K = 4
_STEPS = 5

_NC = 2
_NS = 16
_NW = _NC * _NS
_C = 128
_CH = 80
_EPW = _C * _CH
_EPAD = _NW * _EPW - _E
_NPAD = 10240
_ZR = _NPAD // _NS

_BN = 1000


def _transform_body(h_ref, w_ref, b_ref, out_ref):
  h = h_ref[...]
  for k in range(_K):
    out_ref[k] = jnp.dot(h, w_ref[k], preferred_element_type=jnp.float32) + b_ref[k][None, :]


def _transform(h, lin_W, lin_b):
  return pl.pallas_call(
      _transform_body,
      grid=(_N // _BN,),
      in_specs=[
          pl.BlockSpec((_BN, _D), lambda i: (i, 0)),
          pl.BlockSpec((_K, _D, _D), lambda i: (0, 0, 0)),
          pl.BlockSpec((_K, _D), lambda i: (0, 0)),
      ],
      out_specs=pl.BlockSpec((_K, _BN, _D), lambda i: (0, i, 0)),
      out_shape=jax.ShapeDtypeStruct((_K, _N, _D), jnp.float32),
  )(h, lin_W, lin_b)


def _gru_math(a, h, wih_t, whh_t, bih, bhh):
  gi = jnp.dot(a, wih_t, preferred_element_type=jnp.float32) + bih
  gh = jnp.dot(h, whh_t, preferred_element_type=jnp.float32) + bhh
  r = jax.nn.sigmoid(gi[:, :_D] + gh[:, :_D])
  z = jax.nn.sigmoid(gi[:, _D:2 * _D] + gh[:, _D:2 * _D])
  n = jnp.tanh(gi[:, 2 * _D:] + r * gh[:, 2 * _D:])
  return (1.0 - z) * n + z * h


def _gru_body(a0_ref, a1_ref, h_ref, wih_ref, whh_ref, bih_ref, bhh_ref, out_ref):
  a = a0_ref[...] + a1_ref[...]
  out_ref[...] = _gru_math(a, h_ref[...], wih_ref[...], whh_ref[...],
                           bih_ref[...], bhh_ref[...])


def _gru_tr_body(a0_ref, a1_ref, h_ref, wih_ref, whh_ref, bih_ref, bhh_ref,
                 w_ref, b_ref, out_ref, wh_ref):
  a = a0_ref[...] + a1_ref[...]
  hn = _gru_math(a, h_ref[...], wih_ref[...], whh_ref[...],
                 bih_ref[...], bhh_ref[...])
  out_ref[...] = hn
  for k in range(_K):
    wh_ref[k] = jnp.dot(hn, w_ref[k], preferred_element_type=jnp.float32) + b_ref[k][None, :]


_row_spec = pl.BlockSpec((_BN, _D), lambda i: (i, 0))
_w3_spec = pl.BlockSpec((_D, 3 * _D), lambda i: (0, 0))
_b3_spec = pl.BlockSpec((1, 3 * _D), lambda i: (0, 0))


def _gru(a0, a1, h, wih_t, whh_t, bih, bhh):
  return pl.pallas_call(
      _gru_body,
      grid=(_N // _BN,),
      in_specs=[_row_spec, _row_spec, _row_spec, _w3_spec, _w3_spec, _b3_spec, _b3_spec],
      out_specs=_row_spec,
      out_shape=jax.ShapeDtypeStruct((_N, _D), jnp.float32),
  )(a0, a1, h, wih_t, whh_t, bih, bhh)


def _gru_transform(a0, a1, h, wih_t, whh_t, bih, bhh, lin_W, lin_b):
  return pl.pallas_call(
      _gru_tr_body,
      grid=(_N // _BN,),
      in_specs=[_row_spec, _row_spec, _row_spec, _w3_spec, _w3_spec, _b3_spec, _b3_spec,
                pl.BlockSpec((_K, _D, _D), lambda i: (0, 0, 0)),
                pl.BlockSpec((_K, _D), lambda i: (0, 0))],
      out_specs=[_row_spec, pl.BlockSpec((_K, _BN, _D), lambda i: (0, i, 0))],
      out_shape=[jax.ShapeDtypeStruct((_N, _D), jnp.float32),
                 jax.ShapeDtypeStruct((_K, _N, _D), jnp.float32)],
  )(a0, a1, h, wih_t, whh_t, bih, bhh, lin_W, lin_b)


def _post_body(h0_ref, h1_ref, g0_ref, b0_ref, g1_ref, b1_ref, out_ref):
  eps = 1e-5
  for idx, (h_ref, g_ref, b_ref) in enumerate(
      ((h0_ref, g0_ref, b0_ref), (h1_ref, g1_ref, b1_ref))):
    hv = jnp.maximum(h_ref[...], 0.0)
    mean = jnp.mean(hv, axis=0, keepdims=True)
    var = jnp.mean(jnp.square(hv - mean), axis=0, keepdims=True)
    y = g_ref[...] * (hv - mean) / jnp.sqrt(var + eps) + b_ref[...]
    out_ref[:, idx * _D:(idx + 1) * _D] = y


def _post(h0, h1, g0, b0, g1, b1):
  full = pl.BlockSpec((_N, _D), lambda: (0, 0))
  one = pl.BlockSpec((1, _D), lambda: (0, 0))
  return pl.pallas_call(
      _post_body,
      in_specs=[full, full, one, one, one, one],
      out_specs=pl.BlockSpec((_N, 2 * _D), lambda: (0, 0)),
      out_shape=jax.ShapeDtypeStruct((_N, 2 * _D), jnp.float32),
  )(h0, h1, g0.reshape(1, _D), b0.reshape(1, _D), g1.reshape(1, _D), b1.reshape(1, _D))


def _unpack(packed_v, j, idx_buf, dst_buf):
  for i in range(_C // 16):
    v = packed_v[pl.ds(j * _C + i * 16, 16)]
    dst_buf[pl.ds(i * 16, 16)] = lax.shift_right_logical(v, 16)
    idx_buf[pl.ds(i * 16, 16)] = lax.bitwise_and(v, 0xFFFF)


def _sc_scatter_body(wh_hbm, packed_hbm, zeros_hbm, out_hbm,
                     packed_v, rows_a, rows_b, ia, da, ib, db,
                     acc_sh, sem_z, sem_a, sem_b):
  cid = lax.axis_index("c")
  sid = lax.axis_index("s")
  w = cid * _NS + sid

  zd = pltpu.async_copy(zeros_hbm.at[pl.ds(sid * _ZR, _ZR)],
                        acc_sh.at[pl.ds(sid * _ZR, _ZR)], sem_z)
  pltpu.sync_copy(packed_hbm.at[w], packed_v)
  zd.wait()
  _unpack(packed_v, 0, ia, da)
  plsc.subcore_barrier()

  pltpu.async_copy(wh_hbm.at[ia], rows_a, sem_a)
  _unpack(packed_v, 1, ib, db)

  def body(t, carry):
    c0 = 2 * t
    pltpu.async_copy(wh_hbm.at[ib], rows_b, sem_b)
    pltpu.make_async_copy(wh_hbm.at[ia], rows_a, sem_a).wait()
    pltpu.sync_copy(rows_a, acc_sh.at[da], add=True)
    _unpack(packed_v, c0 + 2, ia, da)
    pltpu.async_copy(wh_hbm.at[ia], rows_a, sem_a)
    pltpu.make_async_copy(wh_hbm.at[ib], rows_b, sem_b).wait()
    pltpu.sync_copy(rows_b, acc_sh.at[db], add=True)
    _unpack(packed_v, c0 + 3, ib, db)
    return carry

  lax.fori_loop(0, _CH // 2 - 1, body, 0)
  pltpu.async_copy(wh_hbm.at[ib], rows_b, sem_b)
  pltpu.make_async_copy(wh_hbm.at[ia], rows_a, sem_a).wait()
  pltpu.sync_copy(rows_a, acc_sh.at[da], add=True)
  pltpu.make_async_copy(wh_hbm.at[ib], rows_b, sem_b).wait()
  pltpu.sync_copy(rows_b, acc_sh.at[db], add=True)
  plsc.subcore_barrier()
  pltpu.sync_copy(acc_sh.at[pl.ds(sid * _ZR, _ZR)],
                  out_hbm.at[pl.ds(cid * _NPAD + sid * _ZR, _ZR)])


@functools.partial(jax.jit, static_argnames=())
def _sc_scatter(wh_flat, packed, zeros):
  mesh = plsc.VectorSubcoreMesh(core_axis_name="c", subcore_axis_name="s")
  return pl.kernel(
      _sc_scatter_body,
      out_type=jax.ShapeDtypeStruct((_NC * _NPAD, _D), jnp.float32),
      mesh=mesh,
      scratch_types=[
          pltpu.VMEM((_EPW,), jnp.int32),
          pltpu.VMEM((_C, _D), jnp.float32),
          pltpu.VMEM((_C, _D), jnp.float32),
          pltpu.VMEM((_C,), jnp.int32),
          pltpu.VMEM((_C,), jnp.int32),
          pltpu.VMEM((_C,), jnp.int32),
          pltpu.VMEM((_C,), jnp.int32),
          pltpu.VMEM_SHARED((_NPAD, _D), jnp.float32),
          pltpu.SemaphoreType.DMA,
          pltpu.SemaphoreType.DMA,
          pltpu.SemaphoreType.DMA,
      ],
  )(wh_flat, packed, zeros)


def _layers_interleaved(x, packed, zeros, p0, p1):
  hs = [x, x]
  whs = [_transform(x, p[0], p[1]) for p in (p0, p1)]
  for step in range(_STEPS):
    for l, p in ((0, p0), (1, p1)):
      parts = _sc_scatter(whs[l].reshape(_K * _N, _D), packed, zeros)
      a0 = parts[:_N]
      a1 = parts[_NPAD:_NPAD + _N]
      if step < _STEPS - 1:
        hs[l], whs[l] = _gru_transform(a0, a1, hs[l], p[2], p[3], p[4], p[5],
                                       p[0], p[1])
      else:
        hs[l] = _gru(a0, a1, hs[l], p[2], p[3], p[4], p[5])
  return hs


def kernel(x, edge_index, etypes,
           lin_W0, lin_b0, gru_Wih0, gru_Whh0, gru_bih0, gru_bhh0, bn_g0, bn_b0,
           lin_W1, lin_b1, gru_Wih1, gru_Whh1, gru_bih1, gru_bhh1, bn_g1, bn_b1):
  src = edge_index[0].astype(jnp.int32)
  dst = edge_index[1].astype(jnp.int32)
  gidx = etypes.astype(jnp.int32) * _N + src
  ar = jnp.arange(_EPAD, dtype=jnp.int32)
  pad_gidx = (ar * 97) % (_K * _N)
  pad_dst = _N + ar % (_NPAD - _N)
  packed = (jnp.concatenate([dst, pad_dst]) << 16) | jnp.concatenate([gidx, pad_gidx])
  packed = packed.reshape(_NW, _EPW)
  zeros = jnp.zeros((_NPAD, _D), jnp.float32)

  p0 = (lin_W0, lin_b0, gru_Wih0.T, gru_Whh0.T,
        gru_bih0.reshape(1, -1), gru_bhh0.reshape(1, -1))
  p1 = (lin_W1, lin_b1, gru_Wih1.T, gru_Whh1.T,
        gru_bih1.reshape(1, -1), gru_bhh1.reshape(1, -1))
  h0, h1 = _layers_interleaved(x, packed, zeros, p0, p1)
  return _post(h0, h1, bn_g0, bn_b0, bn_g1, bn_b1)


# speedup vs baseline: 51.1336x; 1.1087x over previous
import functools

import jax
import jax.numpy as jnp
from jax import lax
from jax.experimental import pallas as pl
from jax.experimental.pallas import tpu as pltpu
from jax.experimental.pallas import tpu_sc as plsc

_N = 10000
_E = 320000
_D = 128
_K = 4
_STEPS = 5

_NC = 2
_NS = 16
_NW = _NC * _NS
_C = 64
_CH = 160
_NBUF = 4
_EPW = _C * _CH
_EPAD = _NW * _EPW - _E
_NPAD = 10240
_ZR = _NPAD // _NS

_BN = 1000


def _transform_body(h_ref, w_ref, b_ref, out_ref):
  h = h_ref[...]
  for k in range(_K):
    out_ref[k] = jnp.dot(h, w_ref[k], preferred_element_type=jnp.float32) + b_ref[k][None, :]


def _transform(h, lin_W, lin_b):
  return pl.pallas_call(
      _transform_body,
      grid=(_N // _BN,),
      in_specs=[
          pl.BlockSpec((_BN, _D), lambda i: (i, 0)),
          pl.BlockSpec((_K, _D, _D), lambda i: (0, 0, 0)),
          pl.BlockSpec((_K, _D), lambda i: (0, 0)),
      ],
      out_specs=pl.BlockSpec((_K, _BN, _D), lambda i: (0, i, 0)),
      out_shape=jax.ShapeDtypeStruct((_K, _N, _D), jnp.float32),
  )(h, lin_W, lin_b)


def _gru_math(a, h, wih_t, whh_t, bih, bhh):
  gi = jnp.dot(a, wih_t, preferred_element_type=jnp.float32) + bih
  gh = jnp.dot(h, whh_t, preferred_element_type=jnp.float32) + bhh
  r = jax.nn.sigmoid(gi[:, :_D] + gh[:, :_D])
  z = jax.nn.sigmoid(gi[:, _D:2 * _D] + gh[:, _D:2 * _D])
  n = jnp.tanh(gi[:, 2 * _D:] + r * gh[:, 2 * _D:])
  return (1.0 - z) * n + z * h


def _gru_body(a0_ref, a1_ref, h_ref, wih_ref, whh_ref, bih_ref, bhh_ref, out_ref):
  a = a0_ref[...] + a1_ref[...]
  out_ref[...] = _gru_math(a, h_ref[...], wih_ref[...], whh_ref[...],
                           bih_ref[...], bhh_ref[...])


def _gru_tr_body(a0_ref, a1_ref, h_ref, wih_ref, whh_ref, bih_ref, bhh_ref,
                 w_ref, b_ref, out_ref, wh_ref):
  a = a0_ref[...] + a1_ref[...]
  hn = _gru_math(a, h_ref[...], wih_ref[...], whh_ref[...],
                 bih_ref[...], bhh_ref[...])
  out_ref[...] = hn
  for k in range(_K):
    wh_ref[k] = jnp.dot(hn, w_ref[k], preferred_element_type=jnp.float32) + b_ref[k][None, :]


_row_spec = pl.BlockSpec((_BN, _D), lambda i: (i, 0))
_w3_spec = pl.BlockSpec((_D, 3 * _D), lambda i: (0, 0))
_b3_spec = pl.BlockSpec((1, 3 * _D), lambda i: (0, 0))


def _gru(a0, a1, h, wih_t, whh_t, bih, bhh):
  return pl.pallas_call(
      _gru_body,
      grid=(_N // _BN,),
      in_specs=[_row_spec, _row_spec, _row_spec, _w3_spec, _w3_spec, _b3_spec, _b3_spec],
      out_specs=_row_spec,
      out_shape=jax.ShapeDtypeStruct((_N, _D), jnp.float32),
  )(a0, a1, h, wih_t, whh_t, bih, bhh)


def _gru_transform(a0, a1, h, wih_t, whh_t, bih, bhh, lin_W, lin_b):
  return pl.pallas_call(
      _gru_tr_body,
      grid=(_N // _BN,),
      in_specs=[_row_spec, _row_spec, _row_spec, _w3_spec, _w3_spec, _b3_spec, _b3_spec,
                pl.BlockSpec((_K, _D, _D), lambda i: (0, 0, 0)),
                pl.BlockSpec((_K, _D), lambda i: (0, 0))],
      out_specs=[_row_spec, pl.BlockSpec((_K, _BN, _D), lambda i: (0, i, 0))],
      out_shape=[jax.ShapeDtypeStruct((_N, _D), jnp.float32),
                 jax.ShapeDtypeStruct((_K, _N, _D), jnp.float32)],
  )(a0, a1, h, wih_t, whh_t, bih, bhh, lin_W, lin_b)


def _post_body(h0_ref, h1_ref, g0_ref, b0_ref, g1_ref, b1_ref, out_ref):
  eps = 1e-5
  for idx, (h_ref, g_ref, b_ref) in enumerate(
      ((h0_ref, g0_ref, b0_ref), (h1_ref, g1_ref, b1_ref))):
    hv = jnp.maximum(h_ref[...], 0.0)
    mean = jnp.mean(hv, axis=0, keepdims=True)
    var = jnp.mean(jnp.square(hv - mean), axis=0, keepdims=True)
    y = g_ref[...] * (hv - mean) / jnp.sqrt(var + eps) + b_ref[...]
    out_ref[:, idx * _D:(idx + 1) * _D] = y


def _post(h0, h1, g0, b0, g1, b1):
  full = pl.BlockSpec((_N, _D), lambda: (0, 0))
  one = pl.BlockSpec((1, _D), lambda: (0, 0))
  return pl.pallas_call(
      _post_body,
      in_specs=[full, full, one, one, one, one],
      out_specs=pl.BlockSpec((_N, 2 * _D), lambda: (0, 0)),
      out_shape=jax.ShapeDtypeStruct((_N, 2 * _D), jnp.float32),
  )(h0, h1, g0.reshape(1, _D), b0.reshape(1, _D), g1.reshape(1, _D), b1.reshape(1, _D))


def _unpack(packed_v, j, idx_buf, dst_buf):
  for i in range(_C // 16):
    v = packed_v[pl.ds(j * _C + i * 16, 16)]
    dst_buf[pl.ds(i * 16, 16)] = lax.shift_right_logical(v, 16)
    idx_buf[pl.ds(i * 16, 16)] = lax.bitwise_and(v, 0xFFFF)


def _sc_scatter_body(wh_hbm, packed_hbm, zeros_hbm, out_hbm, *refs):
  packed_v = refs[0]
  rows = refs[1:1 + _NBUF]
  ibufs = refs[1 + _NBUF:1 + 2 * _NBUF]
  dbufs = refs[1 + 2 * _NBUF:1 + 3 * _NBUF]
  acc_sh = refs[1 + 3 * _NBUF]
  sem_z = refs[2 + 3 * _NBUF]
  sems = refs[3 + 3 * _NBUF:]

  cid = lax.axis_index("c")
  sid = lax.axis_index("s")
  w = cid * _NS + sid

  zd = pltpu.async_copy(zeros_hbm.at[pl.ds(sid * _ZR, _ZR)],
                        acc_sh.at[pl.ds(sid * _ZR, _ZR)], sem_z)
  pltpu.sync_copy(packed_hbm.at[w], packed_v)
  zd.wait()
  plsc.subcore_barrier()

  for b in range(_NBUF):
    _unpack(packed_v, b, ibufs[b], dbufs[b])
    pltpu.async_copy(wh_hbm.at[ibufs[b]], rows[b], sems[b])

  def body(t, carry):
    c = _NBUF * t
    for b in range(_NBUF):
      pltpu.make_async_copy(wh_hbm.at[ibufs[b]], rows[b], sems[b]).wait()
      pltpu.sync_copy(rows[b], acc_sh.at[dbufs[b]], add=True)
      _unpack(packed_v, c + _NBUF + b, ibufs[b], dbufs[b])
      pltpu.async_copy(wh_hbm.at[ibufs[b]], rows[b], sems[b])
    return carry

  lax.fori_loop(0, _CH // _NBUF - 1, body, 0)
  for b in range(_NBUF):
    pltpu.make_async_copy(wh_hbm.at[ibufs[b]], rows[b], sems[b]).wait()
    pltpu.sync_copy(rows[b], acc_sh.at[dbufs[b]], add=True)
  plsc.subcore_barrier()
  pltpu.sync_copy(acc_sh.at[pl.ds(sid * _ZR, _ZR)],
                  out_hbm.at[pl.ds(cid * _NPAD + sid * _ZR, _ZR)])


@functools.partial(jax.jit, static_argnames=())
def _sc_scatter(wh_flat, packed, zeros):
  mesh = plsc.VectorSubcoreMesh(core_axis_name="c", subcore_axis_name="s")
  return pl.kernel(
      _sc_scatter_body,
      out_type=jax.ShapeDtypeStruct((_NC * _NPAD, _D), jnp.float32),
      mesh=mesh,
      scratch_types=(
          [pltpu.VMEM((_EPW,), jnp.int32)]
          + [pltpu.VMEM((_C, _D), jnp.float32)] * _NBUF
          + [pltpu.VMEM((_C,), jnp.int32)] * (2 * _NBUF)
          + [pltpu.VMEM_SHARED((_NPAD, _D), jnp.float32)]
          + [pltpu.SemaphoreType.DMA] * (1 + _NBUF)
      ),
  )(wh_flat, packed, zeros)


def _layers_interleaved(x, packed, zeros, p0, p1):
  hs = [x, x]
  whs = [_transform(x, p[0], p[1]) for p in (p0, p1)]
  for step in range(_STEPS):
    for l, p in ((0, p0), (1, p1)):
      parts = _sc_scatter(whs[l].reshape(_K * _N, _D), packed, zeros)
      a0 = parts[:_N]
      a1 = parts[_NPAD:_NPAD + _N]
      if step < _STEPS - 1:
        hs[l], whs[l] = _gru_transform(a0, a1, hs[l], p[2], p[3], p[4], p[5],
                                       p[0], p[1])
      else:
        hs[l] = _gru(a0, a1, hs[l], p[2], p[3], p[4], p[5])
  return hs


def kernel(x, edge_index, etypes,
           lin_W0, lin_b0, gru_Wih0, gru_Whh0, gru_bih0, gru_bhh0, bn_g0, bn_b0,
           lin_W1, lin_b1, gru_Wih1, gru_Whh1, gru_bih1, gru_bhh1, bn_g1, bn_b1):
  src = edge_index[0].astype(jnp.int32)
  dst = edge_index[1].astype(jnp.int32)
  gidx = etypes.astype(jnp.int32) * _N + src
  ar = jnp.arange(_EPAD, dtype=jnp.int32)
  pad_gidx = (ar * 97) % (_K * _N)
  pad_dst = _N + ar % (_NPAD - _N)
  packed = (jnp.concatenate([dst, pad_dst]) << 16) | jnp.concatenate([gidx, pad_gidx])
  packed = packed.reshape(_NW, _EPW)
  zeros = jnp.zeros((_NPAD, _D), jnp.float32)

  p0 = (lin_W0, lin_b0, gru_Wih0.T, gru_Whh0.T,
        gru_bih0.reshape(1, -1), gru_bhh0.reshape(1, -1))
  p1 = (lin_W1, lin_b1, gru_Wih1.T, gru_Whh1.T,
        gru_bih1.reshape(1, -1), gru_bhh1.reshape(1, -1))
  h0, h1 = _layers_interleaved(x, packed, zeros, p0, p1)
  return _post(h0, h1, bn_g0, bn_b0, bn_g1, bn_b1)


# speedup vs baseline: 51.1626x; 1.0006x over previous
import functools

import jax
import jax.numpy as jnp
from jax import lax
from jax.experimental import pallas as pl
from jax.experimental.pallas import tpu as pltpu
from jax.experimental.pallas import tpu_sc as plsc

_N = 10000
_E = 320000
_D = 128
_K = 4
_STEPS = 5

_NC = 2
_NS = 16
_NW = _NC * _NS
_C = 32
_CH = 320
_NBUF = 8
_EPW = _C * _CH
_EPAD = _NW * _EPW - _E
_NPAD = 10240
_ZR = _NPAD // _NS

_BN = 1000


def _transform_body(h_ref, w_ref, b_ref, out_ref):
  h = h_ref[...]
  for k in range(_K):
    out_ref[k] = jnp.dot(h, w_ref[k], preferred_element_type=jnp.float32) + b_ref[k][None, :]


def _transform(h, lin_W, lin_b):
  return pl.pallas_call(
      _transform_body,
      grid=(_N // _BN,),
      in_specs=[
          pl.BlockSpec((_BN, _D), lambda i: (i, 0)),
          pl.BlockSpec((_K, _D, _D), lambda i: (0, 0, 0)),
          pl.BlockSpec((_K, _D), lambda i: (0, 0)),
      ],
      out_specs=pl.BlockSpec((_K, _BN, _D), lambda i: (0, i, 0)),
      out_shape=jax.ShapeDtypeStruct((_K, _N, _D), jnp.float32),
  )(h, lin_W, lin_b)


def _gru_math(a, h, wih_t, whh_t, bih, bhh):
  gi = jnp.dot(a, wih_t, preferred_element_type=jnp.float32) + bih
  gh = jnp.dot(h, whh_t, preferred_element_type=jnp.float32) + bhh
  r = jax.nn.sigmoid(gi[:, :_D] + gh[:, :_D])
  z = jax.nn.sigmoid(gi[:, _D:2 * _D] + gh[:, _D:2 * _D])
  n = jnp.tanh(gi[:, 2 * _D:] + r * gh[:, 2 * _D:])
  return (1.0 - z) * n + z * h


def _gru_body(a0_ref, a1_ref, h_ref, wih_ref, whh_ref, bih_ref, bhh_ref, out_ref):
  a = a0_ref[...] + a1_ref[...]
  out_ref[...] = _gru_math(a, h_ref[...], wih_ref[...], whh_ref[...],
                           bih_ref[...], bhh_ref[...])


def _gru_tr_body(a0_ref, a1_ref, h_ref, wih_ref, whh_ref, bih_ref, bhh_ref,
                 w_ref, b_ref, out_ref, wh_ref):
  a = a0_ref[...] + a1_ref[...]
  hn = _gru_math(a, h_ref[...], wih_ref[...], whh_ref[...],
                 bih_ref[...], bhh_ref[...])
  out_ref[...] = hn
  for k in range(_K):
    wh_ref[k] = jnp.dot(hn, w_ref[k], preferred_element_type=jnp.float32) + b_ref[k][None, :]


_row_spec = pl.BlockSpec((_BN, _D), lambda i: (i, 0))
_w3_spec = pl.BlockSpec((_D, 3 * _D), lambda i: (0, 0))
_b3_spec = pl.BlockSpec((1, 3 * _D), lambda i: (0, 0))


def _gru(a0, a1, h, wih_t, whh_t, bih, bhh):
  return pl.pallas_call(
      _gru_body,
      grid=(_N // _BN,),
      in_specs=[_row_spec, _row_spec, _row_spec, _w3_spec, _w3_spec, _b3_spec, _b3_spec],
      out_specs=_row_spec,
      out_shape=jax.ShapeDtypeStruct((_N, _D), jnp.float32),
  )(a0, a1, h, wih_t, whh_t, bih, bhh)


def _gru_transform(a0, a1, h, wih_t, whh_t, bih, bhh, lin_W, lin_b):
  return pl.pallas_call(
      _gru_tr_body,
      grid=(_N // _BN,),
      in_specs=[_row_spec, _row_spec, _row_spec, _w3_spec, _w3_spec, _b3_spec, _b3_spec,
                pl.BlockSpec((_K, _D, _D), lambda i: (0, 0, 0)),
                pl.BlockSpec((_K, _D), lambda i: (0, 0))],
      out_specs=[_row_spec, pl.BlockSpec((_K, _BN, _D), lambda i: (0, i, 0))],
      out_shape=[jax.ShapeDtypeStruct((_N, _D), jnp.float32),
                 jax.ShapeDtypeStruct((_K, _N, _D), jnp.float32)],
  )(a0, a1, h, wih_t, whh_t, bih, bhh, lin_W, lin_b)


def _post_body(h0_ref, h1_ref, g0_ref, b0_ref, g1_ref, b1_ref, out_ref):
  eps = 1e-5
  for idx, (h_ref, g_ref, b_ref) in enumerate(
      ((h0_ref, g0_ref, b0_ref), (h1_ref, g1_ref, b1_ref))):
    hv = jnp.maximum(h_ref[...], 0.0)
    mean = jnp.mean(hv, axis=0, keepdims=True)
    var = jnp.mean(jnp.square(hv - mean), axis=0, keepdims=True)
    y = g_ref[...] * (hv - mean) / jnp.sqrt(var + eps) + b_ref[...]
    out_ref[:, idx * _D:(idx + 1) * _D] = y


def _post(h0, h1, g0, b0, g1, b1):
  full = pl.BlockSpec((_N, _D), lambda: (0, 0))
  one = pl.BlockSpec((1, _D), lambda: (0, 0))
  return pl.pallas_call(
      _post_body,
      in_specs=[full, full, one, one, one, one],
      out_specs=pl.BlockSpec((_N, 2 * _D), lambda: (0, 0)),
      out_shape=jax.ShapeDtypeStruct((_N, 2 * _D), jnp.float32),
  )(h0, h1, g0.reshape(1, _D), b0.reshape(1, _D), g1.reshape(1, _D), b1.reshape(1, _D))


def _unpack(packed_v, j, idx_buf, dst_buf):
  for i in range(_C // 16):
    v = packed_v[pl.ds(j * _C + i * 16, 16)]
    dst_buf[pl.ds(i * 16, 16)] = lax.shift_right_logical(v, 16)
    idx_buf[pl.ds(i * 16, 16)] = lax.bitwise_and(v, 0xFFFF)


def _sc_scatter_body(wh_hbm, packed_hbm, zeros_hbm, out_hbm, *refs):
  packed_v = refs[0]
  rows = refs[1:1 + _NBUF]
  ibufs = refs[1 + _NBUF:1 + 2 * _NBUF]
  dbufs = refs[1 + 2 * _NBUF:1 + 3 * _NBUF]
  acc_sh = refs[1 + 3 * _NBUF]
  sem_z = refs[2 + 3 * _NBUF]
  sems = refs[3 + 3 * _NBUF:]

  cid = lax.axis_index("c")
  sid = lax.axis_index("s")
  w = cid * _NS + sid

  zd = pltpu.async_copy(zeros_hbm.at[pl.ds(sid * _ZR, _ZR)],
                        acc_sh.at[pl.ds(sid * _ZR, _ZR)], sem_z)
  pltpu.sync_copy(packed_hbm.at[w], packed_v)
  zd.wait()
  plsc.subcore_barrier()

  for b in range(_NBUF):
    _unpack(packed_v, b, ibufs[b], dbufs[b])
    pltpu.async_copy(wh_hbm.at[ibufs[b]], rows[b], sems[b])

  def body(t, carry):
    c = _NBUF * t
    for b in range(_NBUF):
      pltpu.make_async_copy(wh_hbm.at[ibufs[b]], rows[b], sems[b]).wait()
      pltpu.sync_copy(rows[b], acc_sh.at[dbufs[b]], add=True)
      _unpack(packed_v, c + _NBUF + b, ibufs[b], dbufs[b])
      pltpu.async_copy(wh_hbm.at[ibufs[b]], rows[b], sems[b])
    return carry

  lax.fori_loop(0, _CH // _NBUF - 1, body, 0)
  for b in range(_NBUF):
    pltpu.make_async_copy(wh_hbm.at[ibufs[b]], rows[b], sems[b]).wait()
    pltpu.sync_copy(rows[b], acc_sh.at[dbufs[b]], add=True)
  plsc.subcore_barrier()
  pltpu.sync_copy(acc_sh.at[pl.ds(sid * _ZR, _ZR)],
                  out_hbm.at[pl.ds(cid * _NPAD + sid * _ZR, _ZR)])


@functools.partial(jax.jit, static_argnames=())
def _sc_scatter(wh_flat, packed, zeros):
  mesh = plsc.VectorSubcoreMesh(core_axis_name="c", subcore_axis_name="s")
  return pl.kernel(
      _sc_scatter_body,
      out_type=jax.ShapeDtypeStruct((_NC * _NPAD, _D), jnp.float32),
      mesh=mesh,
      scratch_types=(
          [pltpu.VMEM((_EPW,), jnp.int32)]
          + [pltpu.VMEM((_C, _D), jnp.float32)] * _NBUF
          + [pltpu.VMEM((_C,), jnp.int32)] * (2 * _NBUF)
          + [pltpu.VMEM_SHARED((_NPAD, _D), jnp.float32)]
          + [pltpu.SemaphoreType.DMA] * (1 + _NBUF)
      ),
  )(wh_flat, packed, zeros)


def _layers_interleaved(x, packed, zeros, p0, p1):
  hs = [x, x]
  whs = [_transform(x, p[0], p[1]) for p in (p0, p1)]
  for step in range(_STEPS):
    for l, p in ((0, p0), (1, p1)):
      parts = _sc_scatter(whs[l].reshape(_K * _N, _D), packed, zeros)
      a0 = parts[:_N]
      a1 = parts[_NPAD:_NPAD + _N]
      if step < _STEPS - 1:
        hs[l], whs[l] = _gru_transform(a0, a1, hs[l], p[2], p[3], p[4], p[5],
                                       p[0], p[1])
      else:
        hs[l] = _gru(a0, a1, hs[l], p[2], p[3], p[4], p[5])
  return hs


def kernel(x, edge_index, etypes,
           lin_W0, lin_b0, gru_Wih0, gru_Whh0, gru_bih0, gru_bhh0, bn_g0, bn_b0,
           lin_W1, lin_b1, gru_Wih1, gru_Whh1, gru_bih1, gru_bhh1, bn_g1, bn_b1):
  src = edge_index[0].astype(jnp.int32)
  dst = edge_index[1].astype(jnp.int32)
  gidx = etypes.astype(jnp.int32) * _N + src
  ar = jnp.arange(_EPAD, dtype=jnp.int32)
  pad_gidx = (ar * 97) % (_K * _N)
  pad_dst = _N + ar % (_NPAD - _N)
  packed = (jnp.concatenate([dst, pad_dst]) << 16) | jnp.concatenate([gidx, pad_gidx])
  packed = packed.reshape(_NW, _EPW)
  zeros = jnp.zeros((_NPAD, _D), jnp.float32)

  p0 = (lin_W0, lin_b0, gru_Wih0.T, gru_Whh0.T,
        gru_bih0.reshape(1, -1), gru_bhh0.reshape(1, -1))
  p1 = (lin_W1, lin_b1, gru_Wih1.T, gru_Whh1.T,
        gru_bih1.reshape(1, -1), gru_bhh1.reshape(1, -1))
  h0, h1 = _layers_interleaved(x, packed, zeros, p0, p1)
  return _post(h0, h1, bn_g0, bn_b0, bn_g1, bn_b1)
